# baseline (speedup 1.0000x reference)
import jax
import jax.numpy as jnp
from jax.experimental import pallas as pl
from jax.experimental.pallas import tpu as pltpu

N = 8192
D = 65
BR = 256
NS = 2
NB = N // BR
NSTEP = 2 * NB


def _a_index_map(k):
    def index_map(p, i, k=k):
        s = p * NB + i
        g = ((s + (NS - 1 - k)) // NS) * NS + k
        g = jnp.minimum(g, NSTEP - (NS - k))
        return (g % NB, 0)
    return index_map


def _gcn2_body(x_ref, a0_ref, a1_ref,
               w1_ref, b1_ref, w2_ref, b2_ref,
               out_ref, xw_s, h_s):
    p = pl.program_id(0)
    i = pl.program_id(1)
    s = p * NB + i

    @pl.when((p == 0) & (i == 0))
    def _():
        xw_s[...] = jnp.dot(x_ref[...], w1_ref[...],
                            preferred_element_type=jnp.float32
                            ).astype(jnp.bfloat16)

    @pl.when((p == 1) & (i == 0))
    def _():
        xw_s[...] = jnp.dot(h_s[...], w2_ref[...],
                            preferred_element_type=jnp.float32
                            ).astype(jnp.bfloat16)

    for k, a_ref in enumerate((a0_ref, a1_ref)):
        @pl.when(s % NS == k)
        def _(a_ref=a_ref):
            acc = jnp.dot(a_ref[...].astype(jnp.bfloat16), xw_s[...],
                          preferred_element_type=jnp.float32)

            @pl.when(p == 0)
            def _():
                h_s[pl.ds(i * BR, BR), :] = (
                    jnp.maximum(acc + b1_ref[...], 0.0))

            @pl.when(p == 1)
            def _():
                out_ref[...] = jnp.maximum(acc + b2_ref[...], 0.0)


def _gcn2(X, A, W1, b1r, W2, b2r, interpret=False):
    return pl.pallas_call(
        _gcn2_body,
        grid=(2, NB),
        in_specs=[pl.BlockSpec((N, D), lambda p, i: (0, 0))]
        + [pl.BlockSpec((BR, N), _a_index_map(k)) for k in range(NS)]
        + [
            pl.BlockSpec((D, D), lambda p, i: (0, 0)),
            pl.BlockSpec((1, D), lambda p, i: (0, 0)),
            pl.BlockSpec((D, D), lambda p, i: (0, 0)),
            pl.BlockSpec((1, D), lambda p, i: (0, 0)),
        ],
        out_specs=pl.BlockSpec((BR, D), lambda p, i: (p * i, 0)),
        out_shape=jax.ShapeDtypeStruct((N, D), jnp.float32),
        scratch_shapes=[
            pltpu.VMEM((N, D), jnp.bfloat16),
            pltpu.VMEM((N, D), jnp.float32),
        ],
        interpret=interpret,
    )(X, A, A, W1, b1r, W2, b2r)


def kernel(X, A, W1, b1, W2, b2):
    return _gcn2(X, A, W1, b1.reshape(1, D), W2, b2.reshape(1, D))


# speedup vs baseline: 1.0062x; 1.0062x over previous
import jax
import jax.numpy as jnp
from jax.experimental import pallas as pl
from jax.experimental.pallas import tpu as pltpu

N = 8192
D = 65
BR = 512
NH = N // 2
NB = N // BR


def _gcn2_body(x_ref, a_lo_ref, a_hi_ref,
               w1_ref, b1_ref, w2_ref, b2_ref,
               out_ref, xw_s, h_s):
    p = pl.program_id(0)
    i = pl.program_id(1)

    @pl.when((p == 0) & (i == 0))
    def _():
        xw_s[...] = jnp.dot(x_ref[...], w1_ref[...],
                            preferred_element_type=jnp.float32)

    @pl.when((p == 1) & (i == 0))
    def _():
        xw_s[...] = jnp.dot(h_s[...], w2_ref[...],
                            preferred_element_type=jnp.float32)

    acc = (jnp.dot(a_lo_ref[...], xw_s[:NH, :],
                   preferred_element_type=jnp.float32)
           + jnp.dot(a_hi_ref[...], xw_s[NH:, :],
                     preferred_element_type=jnp.float32))

    @pl.when(p == 0)
    def _():
        h_s[pl.ds(i * BR, BR), :] = jnp.maximum(acc + b1_ref[...], 0.0)

    @pl.when(p == 1)
    def _():
        out_ref[...] = jnp.maximum(acc + b2_ref[...], 0.0)


def _gcn2(X, A, W1, b1r, W2, b2r, interpret=False):
    return pl.pallas_call(
        _gcn2_body,
        grid=(2, NB),
        in_specs=[
            pl.BlockSpec((N, D), lambda p, i: (0, 0)),
            pl.BlockSpec((BR, NH), lambda p, i: (i, 0)),
            pl.BlockSpec((BR, NH), lambda p, i: (i, 1)),
            pl.BlockSpec((D, D), lambda p, i: (0, 0)),
            pl.BlockSpec((1, D), lambda p, i: (0, 0)),
            pl.BlockSpec((D, D), lambda p, i: (0, 0)),
            pl.BlockSpec((1, D), lambda p, i: (0, 0)),
        ],
        out_specs=pl.BlockSpec((BR, D), lambda p, i: (p * i, 0)),
        out_shape=jax.ShapeDtypeStruct((N, D), jnp.float32),
        scratch_shapes=[
            pltpu.VMEM((N, D), jnp.float32),
            pltpu.VMEM((N, D), jnp.float32),
        ],
        interpret=interpret,
    )(X, A, A, W1, b1r, W2, b2r)


def kernel(X, A, W1, b1, W2, b2):
    return _gcn2(X, A, W1, b1.reshape(1, D), W2, b2.reshape(1, D))


# speedup vs baseline: 1.0262x; 1.0199x over previous
import jax
import jax.numpy as jnp
from jax.experimental import pallas as pl
from jax.experimental.pallas import tpu as pltpu

N = 8192
D = 65
BR = 256
NB = N // BR


def _gcn2_body(x_ref, a_ref, w1_ref, b1_ref, w2_ref, b2_ref,
               out_ref, xw_s, h_s):
    p = pl.program_id(0)
    i = pl.program_id(1)

    @pl.when((p == 0) & (i == 0))
    def _():
        xw_s[...] = jnp.dot(x_ref[...], w1_ref[...],
                            preferred_element_type=jnp.float32)

    @pl.when((p == 1) & (i == 0))
    def _():
        xw_s[...] = jnp.dot(h_s[...], w2_ref[...],
                            preferred_element_type=jnp.float32)

    acc = jnp.dot(a_ref[...], xw_s[...], preferred_element_type=jnp.float32)

    @pl.when(p == 0)
    def _():
        h_s[pl.ds(i * BR, BR), :] = jnp.maximum(acc + b1_ref[...], 0.0)

    @pl.when(p == 1)
    def _():
        out_ref[...] = jnp.maximum(acc + b2_ref[...], 0.0)


def _gcn2(X, A, W1, b1r, W2, b2r, interpret=False):
    return pl.pallas_call(
        _gcn2_body,
        grid=(2, NB),
        in_specs=[
            pl.BlockSpec((N, D), lambda p, i: (0, 0)),
            pl.BlockSpec((BR, N), lambda p, i: (i, 0)),
            pl.BlockSpec((D, D), lambda p, i: (0, 0)),
            pl.BlockSpec((1, D), lambda p, i: (0, 0)),
            pl.BlockSpec((D, D), lambda p, i: (0, 0)),
            pl.BlockSpec((1, D), lambda p, i: (0, 0)),
        ],
        out_specs=pl.BlockSpec((BR, D), lambda p, i: (p * i, 0)),
        out_shape=jax.ShapeDtypeStruct((N, D), jnp.float32),
        scratch_shapes=[
            pltpu.VMEM((N, D), jnp.float32),
            pltpu.VMEM((N, D), jnp.float32),
        ],
        interpret=interpret,
    )(X, A, W1, b1r, W2, b2r)


def kernel(X, A, W1, b1, W2, b2):
    return _gcn2(X, A, W1, b1.reshape(1, D), W2, b2.reshape(1, D))
